# Initial kernel scaffold; baseline (speedup 1.0000x reference)
import jax
import jax.numpy as jnp
from jax.experimental import pallas as pl


def kernel(text, text_lengths, table, W1, b1, W2, b2):
    raise NotImplementedError("write your pallas kernel here")


# speedup vs baseline: 10.3041x; 10.3041x over previous
import functools

import jax
import jax.numpy as jnp
from jax import lax
from jax.experimental import pallas as pl
from jax.experimental.pallas import tpu as pltpu
from jax.experimental.pallas import tpu_sc as plsc

_VOCAB = 100000
_HID = 64
_NC = 10
_NCP = 16
_B = 4096
_L = 200
_CHUNK = 100

_NW = 32
_SPW = _B // _NW

_PROJ_BLK = 10000


def _project_body(tab_ref, w1_ref, w2p_ref, p_ref):
    m = jnp.dot(w2p_ref[...], w1_ref[...], preferred_element_type=jnp.float32)
    blk = tab_ref[...]
    p = jnp.dot(blk, m.T, preferred_element_type=jnp.float32)
    rows = lax.broadcasted_iota(jnp.int32, (_PROJ_BLK, _NCP), 0)
    is_row0 = (rows == 0) & (pl.program_id(0) == 0)
    p_ref[...] = jnp.where(is_row0, 0.0, p)


def _project(table, w1, w2p):
    grid = _VOCAB // _PROJ_BLK
    return pl.pallas_call(
        _project_body,
        grid=(grid,),
        in_specs=[
            pl.BlockSpec((_PROJ_BLK, _HID), lambda i: (i, 0)),
            pl.BlockSpec((_HID, _HID), lambda i: (0, 0)),
            pl.BlockSpec((_NCP, _HID), lambda i: (0, 0)),
        ],
        out_specs=pl.BlockSpec((_PROJ_BLK, _NCP), lambda i: (i, 0)),
        out_shape=jax.ShapeDtypeStruct((_VOCAB, _NCP), jnp.float32),
    )(table, w1, w2p)


def _sc_pool_body(p_hbm, text_hbm, out_hbm, idx_v, rows_v, out_v, sem):
    c = lax.axis_index("c")
    s = lax.axis_index("s")
    w = s * 2 + c
    nrow = 2 * _SPW

    pltpu.sync_copy(text_hbm.at[pl.ds(w * nrow, nrow)], idx_v)

    def body(i, carry):
        cp0 = pltpu.async_copy(p_hbm.at[idx_v.at[2 * i]], rows_v.at[0], sem)
        cp1 = pltpu.async_copy(p_hbm.at[idx_v.at[2 * i + 1]], rows_v.at[1], sem)
        cp0.wait()
        cp1.wait()
        accs = [jnp.zeros((_NCP,), jnp.float32) for _ in range(8)]
        nstep = (2 * _CHUNK) // 8

        def inner(r, accs):
            out = []
            for j in range(8):
                k = r * 8 + j
                out.append(accs[j] + rows_v[k // _CHUNK, k % _CHUNK])
            return tuple(out)

        accs = lax.fori_loop(0, nstep, inner, tuple(accs), unroll=True)
        acc = ((accs[0] + accs[1]) + (accs[2] + accs[3])) + (
            (accs[4] + accs[5]) + (accs[6] + accs[7])
        )
        out_v[i] = acc
        return carry

    lax.fori_loop(0, _SPW, body, 0)
    pltpu.sync_copy(out_v, out_hbm.at[pl.ds(w * _SPW, _SPW)])


@functools.partial(
    pl.kernel,
    mesh=plsc.VectorSubcoreMesh(core_axis_name="c", subcore_axis_name="s"),
    out_type=jax.ShapeDtypeStruct((_B, _NCP), jnp.float32),
    compiler_params=pltpu.CompilerParams(use_tc_tiling_on_sc=False),
    scratch_types=[
        pltpu.VMEM((2 * _SPW, _CHUNK), jnp.int32),
        pltpu.VMEM((2, _CHUNK, _NCP), jnp.float32),
        pltpu.VMEM((_SPW, _NCP), jnp.float32),
        pltpu.SemaphoreType.DMA,
    ],
)
def _sc_pool(p_hbm, text_hbm, out_hbm, idx_v, rows_v, out_v, sem):
    _sc_pool_body(p_hbm, text_hbm, out_hbm, idx_v, rows_v, out_v, sem)


def _finalize_body(zsum_ref, b1_ref, w2p_ref, b2p_ref, out_ref):
    bias = (
        jnp.dot(b1_ref[...], w2p_ref[...].T, preferred_element_type=jnp.float32)
        + b2p_ref[...]
    )
    z = zsum_ref[...] * (1.0 / _L) + bias
    cols = lax.broadcasted_iota(jnp.int32, (_B, _NCP), 1)
    z = jnp.where(cols < _NC, z, -1e30)
    m = jnp.max(z, axis=1, keepdims=True)
    e = jnp.exp(z - m)
    lse = jnp.log(jnp.sum(e, axis=1, keepdims=True))
    out = z - m - lse
    out_ref[...] = out[:, :_NC]


def _finalize(zsum, b1r, w2p, b2p):
    return pl.pallas_call(
        _finalize_body,
        out_shape=jax.ShapeDtypeStruct((_B, _NC), jnp.float32),
    )(zsum, b1r, w2p, b2p)


def kernel(text, text_lengths, table, W1, b1, W2, b2):
    del text_lengths
    w2p = jnp.zeros((_NCP, _HID), jnp.float32).at[:_NC].set(W2)
    b2p = jnp.zeros((1, _NCP), jnp.float32).at[0, :_NC].set(b2)
    p = _project(table, W1, w2p)
    zsum = _sc_pool(p, text.reshape(_B * _L // _CHUNK, _CHUNK))
    return _finalize(zsum, b1.reshape(1, _HID), w2p, b2p)


# speedup vs baseline: 19.5116x; 1.8936x over previous
import functools

import jax
import jax.numpy as jnp
from jax import lax
from jax.experimental import pallas as pl
from jax.experimental.pallas import tpu as pltpu
from jax.experimental.pallas import tpu_sc as plsc

_VOCAB = 100000
_VOCABP = 100032
_HID = 64
_NC = 10
_NCP = 16
_B = 4096
_L = 200
_CHUNK = 100

_NW = 32
_SPW = _B // _NW
_CPW = 2 * _SPW
_NBUF = 8

_PACK = _VOCABP // 8
_PBLK = _PACK // 3


def _project_body(tab_ref, w1_ref, w2p_ref, p_ref):
    mt = lax.dot_general(
        w1_ref[...], w2p_ref[...], (((0,), (1,)), ((), ())),
        preferred_element_type=jnp.float32,
    )
    blk3 = tab_ref[...].reshape(_PBLK, 8, _HID)
    cs = [
        jnp.dot(blk3[:, j, :], mt, preferred_element_type=jnp.float32)
        for j in range(8)
    ]
    p = jnp.concatenate(cs, axis=1)
    rows = lax.broadcasted_iota(jnp.int32, (_PBLK, 128), 0)
    lanes = lax.broadcasted_iota(jnp.int32, (_PBLK, 128), 1)
    is_row0 = (rows == 0) & (lanes < _NCP) & (pl.program_id(0) == 0)
    p_ref[...] = jnp.where(is_row0, 0.0, p)


def _project(table, w1, w2p):
    return pl.pallas_call(
        _project_body,
        grid=(3,),
        in_specs=[
            pl.BlockSpec((8 * _PBLK, _HID), lambda i: (i, 0)),
            pl.BlockSpec((_HID, _HID), lambda i: (0, 0)),
            pl.BlockSpec((_NCP, _HID), lambda i: (0, 0)),
        ],
        out_specs=pl.BlockSpec((_PBLK, 128), lambda i: (i, 0)),
        out_shape=jax.ShapeDtypeStruct((_PACK, 128), jnp.float32),
    )(table, w1, w2p)


def _sc_pool_body(p_hbm, text_hbm, out_hbm, idx_v, buf_v, out_v, sem):
    c = lax.axis_index("c")
    s = lax.axis_index("s")
    w = s * 2 + c

    pltpu.sync_copy(text_hbm.at[pl.ds(w * _CPW, _CPW)], idx_v)

    def issue(j, t):
        return pltpu.async_copy(p_hbm.at[idx_v.at[j]], buf_v.at[t], sem.at[t])

    def wait(j, t):
        pltpu.make_async_copy(p_hbm.at[idx_v.at[j]], buf_v.at[t], sem.at[t]).wait()

    def accumulate(t):
        accs = [buf_v[t, r] for r in range(8)]
        for r in range(8, _CHUNK):
            accs[r % 8] = accs[r % 8] + buf_v[t, r]
        return ((accs[0] + accs[1]) + (accs[2] + accs[3])) + (
            (accs[4] + accs[5]) + (accs[6] + accs[7])
        )

    for t in range(_NBUF - 1):
        issue(jnp.int32(t), t)

    def body(i, carry):
        j0 = i * _NBUF
        for t in range(_NBUF):
            j = j0 + t
            jn = j + (_NBUF - 1)

            @pl.when(jn < _CPW)
            def _():
                issue(jn, (t + _NBUF - 1) % _NBUF)

            wait(j, t)
            half = accumulate(t)
            if t % 2 == 0:
                first = half
            else:
                out_v[(j - 1) // 2] = first + half
        return carry

    lax.fori_loop(0, _CPW // _NBUF, body, 0)
    pltpu.sync_copy(out_v, out_hbm.at[pl.ds(w * _SPW, _SPW)])


@functools.partial(
    pl.kernel,
    mesh=plsc.VectorSubcoreMesh(core_axis_name="c", subcore_axis_name="s"),
    out_type=jax.ShapeDtypeStruct((_B, _NCP), jnp.float32),
    compiler_params=pltpu.CompilerParams(use_tc_tiling_on_sc=False),
    scratch_types=[
        pltpu.VMEM((_CPW, _CHUNK), jnp.int32),
        pltpu.VMEM((_NBUF, _CHUNK, _NCP), jnp.float32),
        pltpu.VMEM((_SPW, _NCP), jnp.float32),
        pltpu.SemaphoreType.DMA((_NBUF,)),
    ],
)
def _sc_pool(p_hbm, text_hbm, out_hbm, idx_v, buf_v, out_v, sem):
    _sc_pool_body(p_hbm, text_hbm, out_hbm, idx_v, buf_v, out_v, sem)


def _finalize_body(zsum_ref, b1_ref, w2p_ref, b2p_ref, out_ref):
    bias = (
        jnp.dot(b1_ref[...], w2p_ref[...].T, preferred_element_type=jnp.float32)
        + b2p_ref[...]
    )
    z = zsum_ref[...] * (1.0 / _L) + bias
    cols = lax.broadcasted_iota(jnp.int32, (_B, _NCP), 1)
    z = jnp.where(cols < _NC, z, -1e30)
    m = jnp.max(z, axis=1, keepdims=True)
    e = jnp.exp(z - m)
    lse = jnp.log(jnp.sum(e, axis=1, keepdims=True))
    out = z - m - lse
    out_ref[...] = out[:, :_NC]


def _finalize(zsum, b1r, w2p, b2p):
    return pl.pallas_call(
        _finalize_body,
        out_shape=jax.ShapeDtypeStruct((_B, _NC), jnp.float32),
    )(zsum, b1r, w2p, b2p)


def kernel(text, text_lengths, table, W1, b1, W2, b2):
    del text_lengths
    w2p = jnp.zeros((_NCP, _HID), jnp.float32).at[:_NC].set(W2)
    b2p = jnp.zeros((1, _NCP), jnp.float32).at[0, :_NC].set(b2)
    p_packed = _project(table, W1, w2p)
    p_lin = p_packed.reshape(_VOCABP, _NCP)
    zsum = _sc_pool(p_lin, text.reshape(_B * _L // _CHUNK, _CHUNK))
    return _finalize(zsum, b1.reshape(1, _HID), w2p, b2p)


# speedup vs baseline: 25.6897x; 1.3166x over previous
import functools

import jax
import jax.numpy as jnp
from jax import lax
from jax.experimental import pallas as pl
from jax.experimental.pallas import tpu as pltpu
from jax.experimental.pallas import tpu_sc as plsc

_VOCAB = 100000
_VOCABP = 100352
_HID = 64
_NC = 10
_NCP = 16
_B = 4096
_L = 200
_CHUNK = 100

_NW = 32
_SPW = _B // _NW
_CPW = 2 * _SPW
_NBUF = 8

_PACK = _VOCABP // 8
_PGRID = 8
_LBLK = _VOCABP // _PGRID
_PBLK = _LBLK // 8


def _project_body(tabt_ref, w1_ref, w2p_ref, p_ref):
    mt = lax.dot_general(
        w1_ref[...], w2p_ref[...], (((0,), (1,)), ((), ())),
        preferred_element_type=jnp.float32,
    )
    d = lax.dot_general(
        tabt_ref[...], mt, (((0,), (0,)), ((), ())),
        preferred_element_type=jnp.float32,
    )
    d3 = d.reshape(_PBLK, 8, _NCP)
    p = jnp.concatenate([d3[:, j, :] for j in range(8)], axis=1)
    rows = lax.broadcasted_iota(jnp.int32, (_PBLK, 128), 0)
    lanes = lax.broadcasted_iota(jnp.int32, (_PBLK, 128), 1)
    is_row0 = (rows == 0) & (lanes < _NCP) & (pl.program_id(0) == 0)
    p_ref[...] = jnp.where(is_row0, 0.0, p)


def _project(tabt, w1, w2p):
    return pl.pallas_call(
        _project_body,
        grid=(_PGRID,),
        in_specs=[
            pl.BlockSpec((_HID, _LBLK), lambda i: (0, i)),
            pl.BlockSpec((_HID, _HID), lambda i: (0, 0)),
            pl.BlockSpec((_NCP, _HID), lambda i: (0, 0)),
        ],
        out_specs=pl.BlockSpec((_PBLK, 128), lambda i: (i, 0)),
        out_shape=jax.ShapeDtypeStruct((_PACK, 128), jnp.float32),
    )(tabt, w1, w2p)


def _sc_pool_body(p_hbm, text_hbm, out_hbm, idx_v, buf_v, out_v, sem):
    c = lax.axis_index("c")
    s = lax.axis_index("s")
    w = s * 2 + c

    pltpu.sync_copy(text_hbm.at[pl.ds(w * _CPW, _CPW)], idx_v)

    def issue(j, t):
        return pltpu.async_copy(p_hbm.at[idx_v.at[j]], buf_v.at[t], sem.at[t])

    def wait(j, t):
        pltpu.make_async_copy(p_hbm.at[idx_v.at[j]], buf_v.at[t], sem.at[t]).wait()

    def accumulate(t):
        accs = [buf_v[t, r] for r in range(8)]
        for r in range(8, _CHUNK):
            accs[r % 8] = accs[r % 8] + buf_v[t, r]
        return ((accs[0] + accs[1]) + (accs[2] + accs[3])) + (
            (accs[4] + accs[5]) + (accs[6] + accs[7])
        )

    for t in range(_NBUF - 1):
        issue(jnp.int32(t), t)

    def body(i, carry):
        j0 = i * _NBUF
        for t in range(_NBUF):
            j = j0 + t
            jn = j + (_NBUF - 1)

            @pl.when(jn < _CPW)
            def _():
                issue(jn, (t + _NBUF - 1) % _NBUF)

            wait(j, t)
            half = accumulate(t)
            if t % 2 == 0:
                first = half
            else:
                out_v[(j - 1) // 2] = first + half
        return carry

    lax.fori_loop(0, _CPW // _NBUF, body, 0)
    pltpu.sync_copy(out_v, out_hbm.at[pl.ds(w * _SPW, _SPW)])


@functools.partial(
    pl.kernel,
    mesh=plsc.VectorSubcoreMesh(core_axis_name="c", subcore_axis_name="s"),
    out_type=jax.ShapeDtypeStruct((_B, _NCP), jnp.float32),
    compiler_params=pltpu.CompilerParams(use_tc_tiling_on_sc=False),
    scratch_types=[
        pltpu.VMEM((_CPW, _CHUNK), jnp.int32),
        pltpu.VMEM((_NBUF, _CHUNK, _NCP), jnp.float32),
        pltpu.VMEM((_SPW, _NCP), jnp.float32),
        pltpu.SemaphoreType.DMA((_NBUF,)),
    ],
)
def _sc_pool(p_hbm, text_hbm, out_hbm, idx_v, buf_v, out_v, sem):
    _sc_pool_body(p_hbm, text_hbm, out_hbm, idx_v, buf_v, out_v, sem)


def _finalize_body(zsum_ref, b1_ref, w2p_ref, b2p_ref, out_ref):
    bias = (
        jnp.dot(b1_ref[...], w2p_ref[...].T, preferred_element_type=jnp.float32)
        + b2p_ref[...]
    )
    z = zsum_ref[...] * (1.0 / _L) + bias
    cols = lax.broadcasted_iota(jnp.int32, (_B, _NCP), 1)
    z = jnp.where(cols < _NC, z, -1e30)
    m = jnp.max(z, axis=1, keepdims=True)
    e = jnp.exp(z - m)
    lse = jnp.log(jnp.sum(e, axis=1, keepdims=True))
    out = z - m - lse
    out_ref[...] = out[:, :_NC]


def _finalize(zsum, b1r, w2p, b2p):
    return pl.pallas_call(
        _finalize_body,
        out_shape=jax.ShapeDtypeStruct((_B, _NC), jnp.float32),
    )(zsum, b1r, w2p, b2p)


def kernel(text, text_lengths, table, W1, b1, W2, b2):
    del text_lengths
    w2p = jnp.zeros((_NCP, _HID), jnp.float32).at[:_NC].set(W2)
    b2p = jnp.zeros((1, _NCP), jnp.float32).at[0, :_NC].set(b2)
    p_packed = _project(table.T, W1, w2p)
    p_lin = p_packed.reshape(_VOCABP, _NCP)
    zsum = _sc_pool(p_lin, text.reshape(_B * _L // _CHUNK, _CHUNK))
    return _finalize(zsum, b1.reshape(1, _HID), w2p, b2p)
